# baseline (speedup 1.0000x reference)
import functools

import jax
import jax.numpy as jnp
from jax import lax
from jax.experimental import pallas as pl
from jax.experimental.pallas import tpu as pltpu
from jax.experimental.pallas import tpu_sc as plsc

_NC = 2
_NS = 16


def kernel(x, table):
    B, S, D = x.shape
    NW = _NC * _NS
    rows = S // NW
    mesh = plsc.VectorSubcoreMesh(core_axis_name="c", subcore_axis_name="s")

    @functools.partial(
        pl.kernel,
        out_type=jax.ShapeDtypeStruct((B, S, D), table.dtype),
        mesh=mesh,
        scratch_types=[pltpu.SemaphoreType.DMA],
    )
    def sc_copy(table_hbm, out_hbm, sem):
        wid = lax.axis_index("s") * _NC + lax.axis_index("c")
        base = wid * rows
        copies = [
            pltpu.async_copy(
                table_hbm.at[pl.ds(base, rows)],
                out_hbm.at[b].at[pl.ds(base, rows)],
                sem,
            )
            for b in range(B)
        ]
        for c in copies:
            c.wait()

    return sc_copy(table)


# speedup vs baseline: 54.6387x; 54.6387x over previous
import functools

import jax
import jax.numpy as jnp
from jax import lax
from jax.experimental import pallas as pl
from jax.experimental.pallas import tpu as pltpu
from jax.experimental.pallas import tpu_sc as plsc

_NC = 2
_NS = 16
_CH = 32
_NBUF = 3


def kernel(x, table):
    B, S, D = x.shape
    NW = _NC * _NS
    rows = S // NW
    nchunk = rows // _CH
    mesh = plsc.VectorSubcoreMesh(core_axis_name="c", subcore_axis_name="s")

    @functools.partial(
        pl.kernel,
        out_type=jax.ShapeDtypeStruct((B, S, D), table.dtype),
        mesh=mesh,
        scratch_types=(
            [pltpu.VMEM((_CH, D), table.dtype) for _ in range(_NBUF)]
            + [pltpu.SemaphoreType.DMA, pltpu.SemaphoreType.DMA]
        ),
    )
    def sc_copy(table_hbm, out_hbm, *rest):
        bufs, (in_sem, out_sem) = list(rest[:_NBUF]), rest[_NBUF:]
        wid = lax.axis_index("s") * _NC + lax.axis_index("c")
        base = wid * rows

        def start_in(i):
            return pltpu.async_copy(
                table_hbm.at[pl.ds(base + i * _CH, _CH)],
                bufs[i % _NBUF], in_sem)

        in_copies = [None] * nchunk
        out_copies = [None] * nchunk
        drained = [False] * nchunk
        in_copies[0] = start_in(0)
        for i in range(nchunk):
            in_copies[i].wait()
            out_copies[i] = [
                pltpu.async_copy(
                    bufs[i % _NBUF],
                    out_hbm.at[b].at[pl.ds(base + i * _CH, _CH)],
                    out_sem)
                for b in range(B)
            ]
            if i + 1 < nchunk:
                prev_user = i + 1 - _NBUF
                if prev_user >= 0:
                    for c in out_copies[prev_user]:
                        c.wait()
                    drained[prev_user] = True
                in_copies[i + 1] = start_in(i + 1)
        for i in range(nchunk):
            if not drained[i]:
                for c in out_copies[i]:
                    c.wait()

    return sc_copy(table)
